# baseline (speedup 1.0000x reference)
import functools

import jax
import jax.numpy as jnp
from jax import lax
from jax.experimental import pallas as pl
from jax.experimental.pallas import tpu as pltpu
from jax.experimental.pallas import tpu_sc as plsc

_MARGIN = 1.0
_L = 16

_DNUMS = lax.GatherDimensionNumbers(
    offset_dims=(), collapsed_slice_dims=(0,), start_index_map=(0,))


def _triplet_kernel(T, B, D, NW, TPW):
    mesh = plsc.VectorSubcoreMesh(core_axis_name="c", subcore_axis_name="s")

    @functools.partial(
        pl.kernel,
        mesh=mesh,
        out_type=jax.ShapeDtypeStruct((NW, _L), jnp.float32),
        compiler_params=pltpu.CompilerParams(use_tc_tiling_on_sc=False),
        scratch_types=[
            pltpu.VMEM((TPW,), jnp.int32),
            pltpu.VMEM((TPW,), jnp.int32),
            pltpu.VMEM((TPW,), jnp.int32),
            pltpu.VMEM((2, TPW // 4, D), jnp.float32),
            pltpu.VMEM((2, TPW // 4, D), jnp.float32),
            pltpu.VMEM((2, TPW // 4, D), jnp.float32),
            pltpu.VMEM((_L,), jnp.float32),
            pltpu.SemaphoreType.DMA,
            pltpu.SemaphoreType.DMA,
        ],
    )
    def k(emb_hbm, ia_hbm, ip_hbm, in_hbm, out_hbm, ia_v, ip_v, in_v,
          a_v, p_v, n_v, out_v, sem0, sem1):
        wid = lax.axis_index("s") * 2 + lax.axis_index("c")
        base = wid * TPW
        NCH = 4
        CH = TPW // NCH
        sems = (sem0, sem1)

        pltpu.sync_copy(ia_hbm.at[pl.ds(base, TPW)], ia_v)
        pltpu.sync_copy(ip_hbm.at[pl.ds(base, TPW)], ip_v)
        pltpu.sync_copy(in_hbm.at[pl.ds(base, TPW)], in_v)

        def chunk_copies(c):
            par = c % 2
            sl = pl.ds(c * CH, CH)
            return [
                pltpu.make_async_copy(emb_hbm.at[ia_v.at[sl]],
                                      a_v.at[par], sems[par]),
                pltpu.make_async_copy(emb_hbm.at[ip_v.at[sl]],
                                      p_v.at[par], sems[par]),
                pltpu.make_async_copy(emb_hbm.at[in_v.at[sl]],
                                      n_v.at[par], sems[par]),
            ]

        for cpy in chunk_copies(0):
            cpy.start()

        lanes = lax.iota(jnp.int32, _L)
        perm_idx = {sh: (lanes ^ sh)[:, None] for sh in (1, 2, 4, 8)}

        def permute(v, sh):
            return lax.gather(v, perm_idx[sh], _DNUMS, (1,),
                              mode=lax.GatherScatterMode.PROMISE_IN_BOUNDS)

        G = 8

        def make_group(par):
            def group(g, acc):
                t0 = g * G
                vts = []
                for j in range(G):
                    t = t0 + j
                    lane = None
                    for c in range(D // _L):
                        sl = pl.ds(c * _L, _L)
                        a = a_v[par, t, sl]
                        p = p_v[par, t, sl]
                        n = n_v[par, t, sl]
                        pn = p - n
                        q = (p + n) - a - a
                        lane = pn * q if lane is None else lane + pn * q
                    vts.append(lane)
                sh = 1
                while len(vts) > 1:
                    nxt = []
                    for i in range(0, len(vts), 2):
                        x, y = vts[i], vts[i + 1]
                        mask = (lanes & sh) == 0
                        z = jnp.where(mask, x, y)
                        u = jnp.where(mask, y, x)
                        nxt.append(z + permute(u, sh))
                    vts = nxt
                    sh *= 2
                r = vts[0]
                rr = r + permute(r, 8)
                return acc + jnp.maximum(rr + _MARGIN, 0.0)
            return group

        acc = jnp.zeros((_L,), jnp.float32)
        for c in range(NCH):
            par = c % 2
            if c + 1 < NCH:
                for cpy in chunk_copies(c + 1):
                    cpy.start()
            for cpy in chunk_copies(c):
                cpy.wait()
            acc = lax.fori_loop(0, CH // G, make_group(par), acc)
        acc = acc * 0.5

        for sh in (8, 4, 2, 1):
            acc = acc + permute(acc, sh)
        out_v[...] = acc
        pltpu.sync_copy(out_v, out_hbm.at[wid])

    return k


def kernel(embeddings, target, triplets):
    del target
    T = triplets.shape[0]
    B, D = embeddings.shape
    NW = 32
    TPW = T // NW
    ia = triplets[:, 0]
    ip = triplets[:, 1]
    inn = triplets[:, 2]
    partials = _triplet_kernel(T, B, D, NW, TPW)(embeddings, ia, ip, inn)
    return (jnp.sum(partials[:, 0]) / T, T)


# speedup vs baseline: 1.0255x; 1.0255x over previous
import functools

import jax
import jax.numpy as jnp
from jax import lax
from jax.experimental import pallas as pl
from jax.experimental.pallas import tpu as pltpu
from jax.experimental.pallas import tpu_sc as plsc

_MARGIN = 1.0
_L = 16

_DNUMS = lax.GatherDimensionNumbers(
    offset_dims=(), collapsed_slice_dims=(0,), start_index_map=(0,))


def _stage1_kernel(T, B, D):
    DPT = 4
    TH = T // 2
    NROW = 16
    mesh = plsc.VectorSubcoreMesh(core_axis_name="c", subcore_axis_name="s")

    @functools.partial(
        pl.kernel,
        mesh=mesh,
        out_type=jax.ShapeDtypeStruct((NROW * T,), jnp.float32),
        compiler_params=pltpu.CompilerParams(use_tc_tiling_on_sc=False, needs_layout_passes=False),
        scratch_types=[
            pltpu.VMEM((DPT * B,), jnp.float32),
            pltpu.VMEM((TH,), jnp.int32),
            pltpu.VMEM((TH,), jnp.int32),
            pltpu.VMEM((TH,), jnp.int32),
            pltpu.VMEM((TH,), jnp.float32),
            pltpu.SemaphoreType.DMA,
        ],
    )
    def k(emb_hbm, ia_hbm, ip_hbm, in_hbm, out_hbm, rows_v, ia_v, ip_v,
          in_v, part_v, sem):
        c = lax.axis_index("c")
        s = lax.axis_index("s")
        dg = s % 8
        half = s // 8
        d0 = c * (D // 2) + dg * DPT
        tb = half * TH

        copies = [
            pltpu.make_async_copy(
                emb_hbm.at[pl.ds((d0 + i) * B, B)],
                rows_v.at[pl.ds(i * B, B)], sem)
            for i in range(DPT)
        ] + [
            pltpu.make_async_copy(h.at[pl.ds(tb, TH)], v, sem)
            for h, v in ((ia_hbm, ia_v), (ip_hbm, ip_v), (in_hbm, in_v))
        ]
        for cpy in copies:
            cpy.start()
        for cpy in copies:
            cpy.wait()

        def body(g, carry):
            sl = pl.ds(g * _L, _L)
            iav = ia_v[sl]
            ipv = ip_v[sl]
            inv = in_v[sl]
            contrib = None
            for i in range(DPT):
                if i == 0:
                    ja, jp, jn = iav, ipv, inv
                else:
                    ja = iav + (i * B)
                    jp = ipv + (i * B)
                    jn = inv + (i * B)
                a = plsc.load_gather(rows_v, [ja])
                p = plsc.load_gather(rows_v, [jp])
                n = plsc.load_gather(rows_v, [jn])
                m = (p - n) * ((p + n) - a - a)
                contrib = m if contrib is None else contrib + m
            part_v[sl] = contrib
            return carry

        lax.fori_loop(0, TH // _L, body, jnp.int32(0))

        row = c * 8 + dg
        pltpu.sync_copy(part_v, out_hbm.at[pl.ds(row * T + tb, TH)])

    return k


def _stage2_kernel(T):
    NW = 32
    NROW = 16
    TPW = T // NW
    mesh = plsc.VectorSubcoreMesh(core_axis_name="c", subcore_axis_name="s")

    @functools.partial(
        pl.kernel,
        mesh=mesh,
        out_type=jax.ShapeDtypeStruct((NW, _L), jnp.float32),
        compiler_params=pltpu.CompilerParams(use_tc_tiling_on_sc=False, needs_layout_passes=False),
        scratch_types=[
            pltpu.VMEM((NROW, TPW), jnp.float32),
            pltpu.VMEM((_L,), jnp.float32),
            pltpu.SemaphoreType.DMA,
        ],
    )
    def k(sp_hbm, out_hbm, srows_v, out_v, sem):
        wid = lax.axis_index("s") * 2 + lax.axis_index("c")
        base = wid * TPW

        copies = [
            pltpu.make_async_copy(sp_hbm.at[pl.ds(r * T + base, TPW)],
                                  srows_v.at[r], sem)
            for r in range(NROW)
        ]
        for cpy in copies:
            cpy.start()
        for cpy in copies:
            cpy.wait()

        lanes = lax.iota(jnp.int32, _L)
        perm_idx = {sh: (lanes ^ sh)[:, None] for sh in (1, 2, 4, 8)}

        def permute(v, sh):
            return lax.gather(v, perm_idx[sh], _DNUMS, (1,),
                              mode=lax.GatherScatterMode.PROMISE_IN_BOUNDS)

        def body(g, acc):
            sl = pl.ds(g * _L, _L)
            v = None
            for r in range(NROW):
                x = srows_v[r, sl]
                v = x if v is None else v + x
            return acc + jnp.maximum(v + _MARGIN, 0.0)

        acc = lax.fori_loop(0, TPW // _L, body,
                            jnp.zeros((_L,), jnp.float32))
        for sh in (8, 4, 2, 1):
            acc = acc + permute(acc, sh)
        out_v[...] = acc
        pltpu.sync_copy(out_v, out_hbm.at[wid])

    return k


def kernel(embeddings, target, triplets):
    del target
    T = triplets.shape[0]
    B, D = embeddings.shape
    emb_flat = embeddings.T.reshape(-1)
    ia = triplets[:, 0]
    ip = triplets[:, 1]
    inn = triplets[:, 2]
    s_part = _stage1_kernel(T, B, D)(emb_flat, ia, ip, inn)
    out = _stage2_kernel(T)(s_part)
    return (jnp.sum(out[:, 0]) / T, T)


# speedup vs baseline: 1.0658x; 1.0393x over previous
import functools

import jax
import jax.numpy as jnp
from jax import lax
from jax.experimental import pallas as pl
from jax.experimental.pallas import tpu as pltpu
from jax.experimental.pallas import tpu_sc as plsc

_MARGIN = 1.0
_L = 16

_DNUMS = lax.GatherDimensionNumbers(
    offset_dims=(), collapsed_slice_dims=(0,), start_index_map=(0,))


def _loss_kernel(T, B, D):
    DPT = 4
    TH = T // 2
    NW = 32
    mesh = plsc.VectorSubcoreMesh(core_axis_name="c", subcore_axis_name="s")

    @functools.partial(
        pl.kernel,
        mesh=mesh,
        out_type=jax.ShapeDtypeStruct((NW, _L), jnp.float32),
        compiler_params=pltpu.CompilerParams(
            use_tc_tiling_on_sc=False, needs_layout_passes=False),
        scratch_types=[
            pltpu.VMEM((DPT * B,), jnp.float32),
            pltpu.VMEM((TH,), jnp.int32),
            pltpu.VMEM((TH,), jnp.int32),
            pltpu.VMEM((TH,), jnp.int32),
            pltpu.VMEM((TH // 128, 128), jnp.float32),
            pltpu.VMEM((TH // 128,), jnp.int32),
            pltpu.VMEM((DPT, 128), jnp.float32),
            pltpu.VMEM((_L,), jnp.float32),
            pltpu.VMEM_SHARED((TH // 128, 128), jnp.float32),
            pltpu.SemaphoreType.DMA,
        ],
    )
    def k(emb_hbm, ia_hbm, ip_hbm, in_hbm, out_hbm, rows_v, ia_v, ip_v,
          in_v, part_v, sidx_v, sl_v, out_v, acc_sh, sem):
        c = lax.axis_index("c")
        s = lax.axis_index("s")
        d0 = s * DPT
        tb = c * TH
        NR = TH // 128

        copies = [
            pltpu.make_async_copy(
                emb_hbm.at[pl.ds((d0 + i) * B, B)],
                rows_v.at[pl.ds(i * B, B)], sem)
            for i in range(DPT)
        ] + [
            pltpu.make_async_copy(h.at[pl.ds(tb, TH)], v, sem)
            for h, v in ((ia_hbm, ia_v), (ip_hbm, ip_v), (in_hbm, in_v))
        ]
        for cpy in copies:
            cpy.start()

        lanes = lax.iota(jnp.int32, _L)
        for kk in range(NR // _L):
            sidx_v[pl.ds(kk * _L, _L)] = lanes + (kk * _L)

        for cpy in copies:
            cpy.wait()

        def one_group(g):
            sl = pl.ds(g * _L, _L)
            iav = ia_v[sl]
            ipv = ip_v[sl]
            inv = in_v[sl]
            contrib = None
            for i in range(DPT):
                if i == 0:
                    ja, jp, jn = iav, ipv, inv
                else:
                    ja = iav + (i * B)
                    jp = ipv + (i * B)
                    jn = inv + (i * B)
                a = plsc.load_gather(rows_v, [ja])
                p = plsc.load_gather(rows_v, [jp])
                n = plsc.load_gather(rows_v, [jn])
                m = (p - n) * ((p + n) - a - a)
                contrib = m if contrib is None else contrib + m
            part_v[g >> 3, pl.ds((g & 7) * _L, _L)] = contrib

        def body(gi, carry):
            one_group(2 * gi)
            one_group(2 * gi + 1)
            return carry

        lax.fori_loop(0, TH // (2 * _L), body, jnp.int32(0))

        @pl.when(s == 0)
        def _():
            pltpu.sync_copy(part_v, acc_sh)
        plsc.subcore_barrier()

        @pl.when(s != 0)
        def _():
            pltpu.sync_copy(part_v, acc_sh.at[sidx_v], add=True)
        plsc.subcore_barrier()

        pltpu.sync_copy(acc_sh.at[pl.ds(s * DPT, DPT)], sl_v)

        perm_idx = {sh: (lanes ^ sh)[:, None] for sh in (1, 2, 4, 8)}

        def permute(v, sh):
            return lax.gather(v, perm_idx[sh], _DNUMS, (1,),
                              mode=lax.GatherScatterMode.PROMISE_IN_BOUNDS)

        acc = jnp.zeros((_L,), jnp.float32)
        for row in range(DPT):
            for kk in range(128 // _L):
                v = sl_v[row, pl.ds(kk * _L, _L)]
                acc = acc + jnp.maximum(v + _MARGIN, 0.0)
        for sh in (8, 4, 2, 1):
            acc = acc + permute(acc, sh)
        out_v[...] = acc
        wid = s * 2 + c
        pltpu.sync_copy(out_v, out_hbm.at[wid])

    return k


def kernel(embeddings, target, triplets):
    del target
    T = triplets.shape[0]
    B, D = embeddings.shape
    emb_flat = embeddings.T.reshape(-1)
    ia = triplets[:, 0]
    ip = triplets[:, 1]
    inn = triplets[:, 2]
    out = _loss_kernel(T, B, D)(emb_flat, ia, ip, inn)
    return (jnp.sum(out[:, 0]) / T, T)


# speedup vs baseline: 1.1641x; 1.0923x over previous
import functools

import jax
import jax.numpy as jnp
from jax import lax
from jax.experimental import pallas as pl
from jax.experimental.pallas import tpu as pltpu
from jax.experimental.pallas import tpu_sc as plsc

_MARGIN = 1.0
_L = 16

_DNUMS = lax.GatherDimensionNumbers(
    offset_dims=(), collapsed_slice_dims=(0,), start_index_map=(0,))


def _loss_kernel(T, B, D):
    DPT = 4
    TH = T // 2
    NW = 32
    NTR = B // 128
    mesh = plsc.VectorSubcoreMesh(core_axis_name="c", subcore_axis_name="s")

    @functools.partial(
        pl.kernel,
        mesh=mesh,
        out_type=jax.ShapeDtypeStruct((NW, _L), jnp.float32),
        compiler_params=pltpu.CompilerParams(
            use_tc_tiling_on_sc=False, needs_layout_passes=False),
        scratch_types=[
            pltpu.VMEM((DPT * NTR, 128), jnp.float32),
            pltpu.VMEM((DPT, NTR), jnp.int32),
            pltpu.VMEM((TH,), jnp.int32),
            pltpu.VMEM((TH,), jnp.int32),
            pltpu.VMEM((TH,), jnp.int32),
            pltpu.VMEM((TH // 128, 128), jnp.float32),
            pltpu.VMEM((TH // 128,), jnp.int32),
            pltpu.VMEM((DPT, 128), jnp.float32),
            pltpu.VMEM((_L,), jnp.float32),
            pltpu.VMEM_SHARED((TH // 128, 128), jnp.float32),
            pltpu.SemaphoreType.DMA,
        ],
    )
    def k(emb_hbm, ia_hbm, ip_hbm, in_hbm, out_hbm, rows_v, gidx_v, ia_v,
          ip_v, in_v, part_v, sidx_v, sl_v, out_v, acc_sh, sem):
        c = lax.axis_index("c")
        s = lax.axis_index("s")
        d0 = s * DPT
        tb = c * TH
        NR = TH // 128

        idx_copies = [
            pltpu.make_async_copy(h.at[pl.ds(tb, TH)], v, sem)
            for h, v in ((ia_hbm, ia_v), (ip_hbm, ip_v), (in_hbm, in_v))
        ]
        for cpy in idx_copies:
            cpy.start()

        lanes = lax.iota(jnp.int32, _L)
        lanes8 = lanes * 8
        for i in range(DPT):
            d = d0 + i
            base = (d >> 3) * (8 * NTR) + (d & 7)
            for kk in range(NTR // _L):
                gidx_v[i, pl.ds(kk * _L, _L)] = lanes8 + (base + 128 * kk)

        row_copies = [
            pltpu.make_async_copy(emb_hbm.at[gidx_v.at[i]],
                                  rows_v.at[pl.ds(i * NTR, NTR)], sem)
            for i in range(DPT)
        ]
        for cpy in row_copies:
            cpy.start()

        for kk in range(NR // _L):
            sidx_v[pl.ds(kk * _L, _L)] = lanes + (kk * _L)

        for cpy in idx_copies:
            cpy.wait()
        for cpy in row_copies:
            cpy.wait()

        def one_group(g):
            sl = pl.ds(g * _L, _L)
            iav = ia_v[sl]
            ipv = ip_v[sl]
            inv = in_v[sl]
            his = [v >> 7 for v in (iav, ipv, inv)]
            los = [v & 127 for v in (iav, ipv, inv)]
            contrib = None
            for i in range(DPT):
                if i == 0:
                    rows = his
                else:
                    rows = [h + (i * NTR) for h in his]
                a = plsc.load_gather(rows_v, [rows[0], los[0]])
                p = plsc.load_gather(rows_v, [rows[1], los[1]])
                n = plsc.load_gather(rows_v, [rows[2], los[2]])
                m = (p - n) * ((p + n) - a - a)
                contrib = m if contrib is None else contrib + m
            part_v[g >> 3, pl.ds((g & 7) * _L, _L)] = contrib

        UN = 4
        def body(gi, carry):
            for u in range(UN):
                one_group(UN * gi + u)
            return carry

        lax.fori_loop(0, TH // (UN * _L), body, jnp.int32(0))

        @pl.when(s == 0)
        def _():
            pltpu.sync_copy(part_v, acc_sh)
        plsc.subcore_barrier()

        @pl.when(s != 0)
        def _():
            pltpu.sync_copy(part_v, acc_sh.at[sidx_v], add=True)
        plsc.subcore_barrier()

        pltpu.sync_copy(acc_sh.at[pl.ds(s * DPT, DPT)], sl_v)

        perm_idx = {sh: (lanes ^ sh)[:, None] for sh in (1, 2, 4, 8)}

        def permute(v, sh):
            return lax.gather(v, perm_idx[sh], _DNUMS, (1,),
                              mode=lax.GatherScatterMode.PROMISE_IN_BOUNDS)

        acc = jnp.zeros((_L,), jnp.float32)
        for row in range(DPT):
            for kk in range(128 // _L):
                v = sl_v[row, pl.ds(kk * _L, _L)]
                acc = acc + jnp.maximum(v + _MARGIN, 0.0)
        for sh in (8, 4, 2, 1):
            acc = acc + permute(acc, sh)
        out_v[...] = acc
        wid = s * 2 + c
        pltpu.sync_copy(out_v, out_hbm.at[wid])

    return k


def kernel(embeddings, target, triplets):
    del target
    T = triplets.shape[0]
    B, D = embeddings.shape
    emb_tiles = (embeddings.T.reshape(D // 8, 8, B // 128, 128)
                 .transpose(0, 2, 1, 3).reshape((B * D) // 128, 128))
    ia = triplets[:, 0]
    ip = triplets[:, 1]
    inn = triplets[:, 2]
    out = _loss_kernel(T, B, D)(emb_tiles, ia, ip, inn)
    return (jnp.sum(out[:, 0]) / T, T)


# speedup vs baseline: 1.2276x; 1.0545x over previous
import functools

import jax
import jax.numpy as jnp
from jax import lax
from jax.experimental import pallas as pl
from jax.experimental.pallas import tpu as pltpu
from jax.experimental.pallas import tpu_sc as plsc

_MARGIN = 1.0
_L = 16

_DNUMS = lax.GatherDimensionNumbers(
    offset_dims=(), collapsed_slice_dims=(0,), start_index_map=(0,))


def _loss_kernel(T, B, D):
    DPT = 4
    TH = T // 2
    NW = 32
    NTR = B // 128
    NJ = TH // 128
    mesh = plsc.VectorSubcoreMesh(core_axis_name="c", subcore_axis_name="s")

    @functools.partial(
        pl.kernel,
        mesh=mesh,
        out_type=jax.ShapeDtypeStruct((NW, _L), jnp.float32),
        compiler_params=pltpu.CompilerParams(
            use_tc_tiling_on_sc=False, needs_layout_passes=False),
        scratch_types=[
            pltpu.VMEM((DPT * NTR, 128), jnp.float32),
            pltpu.VMEM((DPT, NTR), jnp.int32),
            pltpu.VMEM((3, NJ), jnp.int32),
            pltpu.VMEM((NJ, 128), jnp.int32),
            pltpu.VMEM((NJ, 128), jnp.int32),
            pltpu.VMEM((NJ, 128), jnp.int32),
            pltpu.VMEM((NJ, 128), jnp.float32),
            pltpu.VMEM((NJ,), jnp.int32),
            pltpu.VMEM((DPT, 128), jnp.float32),
            pltpu.VMEM((_L,), jnp.float32),
            pltpu.VMEM_SHARED((NJ, 128), jnp.float32),
            pltpu.SemaphoreType.DMA,
        ],
    )
    def k(emb_hbm, trip_hbm, out_hbm, rows_v, gidx_v, tidx_v, ia_v, ip_v,
          in_v, part_v, sidx_v, sl_v, out_v, acc_sh, sem):
        c = lax.axis_index("c")
        s = lax.axis_index("s")
        d0 = s * DPT

        lanes = lax.iota(jnp.int32, _L)
        lanes4 = lanes * 4
        lanes8 = lanes * 8

        for r in range(3):
            base_r = c * (4 * NJ) + r
            for kk in range(NJ // _L):
                tidx_v[r, pl.ds(kk * _L, _L)] = lanes4 + (base_r + 64 * kk)
        trip_copies = [
            pltpu.make_async_copy(trip_hbm.at[tidx_v.at[r]], v, sem)
            for r, v in ((0, ia_v), (1, ip_v), (2, in_v))
        ]
        for cpy in trip_copies:
            cpy.start()

        for i in range(DPT):
            d = d0 + i
            base = (d >> 3) * (8 * NTR) + (d & 7)
            for kk in range(NTR // _L):
                gidx_v[i, pl.ds(kk * _L, _L)] = lanes8 + (base + 128 * kk)
        row_copies = [
            pltpu.make_async_copy(emb_hbm.at[gidx_v.at[i]],
                                  rows_v.at[pl.ds(i * NTR, NTR)], sem)
            for i in range(DPT)
        ]
        for cpy in row_copies:
            cpy.start()

        for kk in range(NJ // _L):
            sidx_v[pl.ds(kk * _L, _L)] = lanes + (kk * _L)

        for cpy in trip_copies:
            cpy.wait()
        for cpy in row_copies:
            cpy.wait()

        def one_group(g):
            j = g >> 3
            co = (g & 7) * _L
            sl = pl.ds(co, _L)
            iav = ia_v[j, sl]
            ipv = ip_v[j, sl]
            inv = in_v[j, sl]
            his = [v >> 7 for v in (iav, ipv, inv)]
            los = [v & 127 for v in (iav, ipv, inv)]
            contrib = None
            for i in range(DPT):
                if i == 0:
                    rows = his
                else:
                    rows = [h + (i * NTR) for h in his]
                a = plsc.load_gather(rows_v, [rows[0], los[0]])
                p = plsc.load_gather(rows_v, [rows[1], los[1]])
                n = plsc.load_gather(rows_v, [rows[2], los[2]])
                m = (p - n) * ((p + n) - a - a)
                contrib = m if contrib is None else contrib + m
            part_v[j, sl] = contrib

        UN = 2
        def body(gi, carry):
            for u in range(UN):
                one_group(UN * gi + u)
            return carry

        lax.fori_loop(0, TH // (UN * _L), body, jnp.int32(0))

        @pl.when(s == 0)
        def _():
            pltpu.sync_copy(part_v, acc_sh)
        plsc.subcore_barrier()

        @pl.when(s != 0)
        def _():
            pltpu.sync_copy(part_v, acc_sh.at[sidx_v], add=True)
        plsc.subcore_barrier()

        pltpu.sync_copy(acc_sh.at[pl.ds(s * DPT, DPT)], sl_v)

        perm_idx = {sh: (lanes ^ sh)[:, None] for sh in (1, 2, 4, 8)}

        def permute(v, sh):
            return lax.gather(v, perm_idx[sh], _DNUMS, (1,),
                              mode=lax.GatherScatterMode.PROMISE_IN_BOUNDS)

        acc = jnp.zeros((_L,), jnp.float32)
        for row in range(DPT):
            for kk in range(128 // _L):
                v = sl_v[row, pl.ds(kk * _L, _L)]
                acc = acc + jnp.maximum(v + _MARGIN, 0.0)
        for sh in (8, 4, 2, 1):
            acc = acc + permute(acc, sh)
        out_v[...] = acc
        wid = s * 2 + c
        pltpu.sync_copy(out_v, out_hbm.at[wid])

    return k


def kernel(embeddings, target, triplets):
    del target
    T = triplets.shape[0]
    B, D = embeddings.shape
    emb_tiles = (embeddings.T.reshape(D // 8, 8, B // 128, 128)
                 .transpose(0, 2, 1, 3).reshape((B * D) // 128, 128))
    trip_tiles = (jnp.pad(triplets.T, ((0, 1), (0, 0)))
                  .reshape(4, T // 128, 128).transpose(1, 0, 2)
                  .reshape((T * 4) // 128, 128))
    out = _loss_kernel(T, B, D)(emb_tiles, trip_tiles)
    return (jnp.sum(out[:, 0]) / T, T)
